# Initial kernel scaffold; baseline (speedup 1.0000x reference)
import functools
import jax
import jax.numpy as jnp
from jax import lax
from jax.experimental import pallas as pl
from jax.experimental.pallas import tpu as pltpu
from jax.experimental.pallas import tpu_sc as plsc

N = 10000
E = 320000
NS = 16
EPT = E // NS
G = 80
NCHUNK = EPT // G
ROWS_PT = N // NS
ZR = 125


def _conv_body(F, x0_hbm, x1_hbm, col_hbm, row_hbm, w_hbm, y0_hbm, y1_hbm,
               col_v, row_v, w_v, rows_v, zero_v, acc_sh, sem):
    c = lax.axis_index("c")
    s = lax.axis_index("s")

    pltpu.sync_copy(col_hbm.at[s], col_v)
    pltpu.sync_copy(row_hbm.at[s], row_v)
    pltpu.sync_copy(w_hbm.at[s], w_v)

    def zbody(i, carry):
        for k in range(F // 16):
            zero_v[i, pl.ds(k * 16, 16)] = jnp.zeros((16,), jnp.float32)
        return carry
    lax.fori_loop(0, ZR, zbody, 0)
    for rr in range(ROWS_PT // ZR):
        pltpu.sync_copy(zero_v, acc_sh.at[pl.ds(s * ROWS_PT + rr * ZR, ZR)])
    plsc.subcore_barrier()

    def chunk_body(i, carry):
        @pl.when(c == 0)
        def _():
            pltpu.async_copy(x0_hbm.at[col_v.at[i]], rows_v, sem).wait()

        @pl.when(c == 1)
        def _():
            pltpu.async_copy(x1_hbm.at[col_v.at[i]], rows_v, sem).wait()

        for j in range(G):
            wj = plsc.load_gather(
                w_v, [jnp.full((16,), i, jnp.int32), jnp.full((16,), j, jnp.int32)])
            for k in range(F // 16):
                rows_v[j, pl.ds(k * 16, 16)] = rows_v[j, pl.ds(k * 16, 16)] * wj

        pltpu.sync_copy(rows_v, acc_sh.at[row_v.at[i]], add=True)
        return carry
    lax.fori_loop(0, NCHUNK, chunk_body, 0)

    plsc.subcore_barrier()

    @pl.when(c == 0)
    def _():
        pltpu.sync_copy(acc_sh.at[pl.ds(s * ROWS_PT, ROWS_PT)],
                        y0_hbm.at[pl.ds(s * ROWS_PT, ROWS_PT)])

    @pl.when(c == 1)
    def _():
        pltpu.sync_copy(acc_sh.at[pl.ds(s * ROWS_PT, ROWS_PT)],
                        y1_hbm.at[pl.ds(s * ROWS_PT, ROWS_PT)])


@functools.lru_cache(maxsize=None)
def _make_conv(F):
    mesh = plsc.VectorSubcoreMesh(core_axis_name="c", subcore_axis_name="s")
    return pl.kernel(
        functools.partial(_conv_body, F),
        out_type=(jax.ShapeDtypeStruct((N, F), jnp.float32),
                  jax.ShapeDtypeStruct((N, F), jnp.float32)),
        mesh=mesh,
        scratch_types=[
            pltpu.VMEM((NCHUNK, G), jnp.int32),
            pltpu.VMEM((NCHUNK, G), jnp.int32),
            pltpu.VMEM((NCHUNK, G), jnp.float32),
            pltpu.VMEM((G, F), jnp.float32),
            pltpu.VMEM((ZR, F), jnp.float32),
            pltpu.VMEM_SHARED((N, F), jnp.float32),
            pltpu.SemaphoreType.DMA,
        ],
    )


BN = 1000


def _gates_body(p0, p1, q0, q1, s0, s1, wgT, wcT, bg, bc, z_o, rh_o, cp_o):
    d4 = 2.0 * s0[...] - p0[...]
    d5 = 2.0 * s1[...] - p1[...]
    g = (jnp.dot(p0[...], wgT[0:128, :], preferred_element_type=jnp.float32)
         + jnp.dot(p1[...], wgT[128:256, :], preferred_element_type=jnp.float32)
         + jnp.dot(q0[...], wgT[256:384, :], preferred_element_type=jnp.float32)
         + jnp.dot(q1[...], wgT[384:512, :], preferred_element_type=jnp.float32)
         + jnp.dot(d4, wgT[512:640, :], preferred_element_type=jnp.float32)
         + jnp.dot(d5, wgT[640:768, :], preferred_element_type=jnp.float32)
         + bg[...])
    g = jax.nn.sigmoid(g)
    z = g[:, 0:128]
    r = g[:, 128:256]
    z_o[...] = z
    rh_o[...] = r * p1[...]
    cp_o[...] = (jnp.dot(p0[...], wcT[0:128, :], preferred_element_type=jnp.float32)
                 + jnp.dot(q0[...], wcT[256:384, :], preferred_element_type=jnp.float32)
                 + jnp.dot(d4, wcT[512:640, :], preferred_element_type=jnp.float32)
                 + bc[...])


def _final_body(z, cp, rh, t1a, t1b, u1a, u1b, hx, wcT, bc, h_o):
    rhv = rh[...]
    cand = (cp[...]
            + jnp.dot(rhv, wcT[128:256, :], preferred_element_type=jnp.float32)
            + jnp.dot(t1a[...], wcT[384:448, :], preferred_element_type=jnp.float32)
            + jnp.dot(t1b[...], wcT[448:512, :], preferred_element_type=jnp.float32)
            + jnp.dot(2.0 * u1a[...] - rhv[:, 0:64], wcT[640:704, :],
                      preferred_element_type=jnp.float32)
            + jnp.dot(2.0 * u1b[...] - rhv[:, 64:128], wcT[704:768, :],
                      preferred_element_type=jnp.float32))
    cand = jnp.tanh(cand)
    zv = z[...]
    h_o[...] = (1.0 - zv) * hx[...] + zv * cand


def _blk(fw):
    return pl.BlockSpec((BN, fw), lambda i: (i, 0))


def _full(shape):
    return pl.BlockSpec(shape, lambda i: tuple(0 for _ in shape))


def _tc_gates(p0, p1, q0, q1, s0, s1, wgT, wcT, bg, bc):
    grid = (N // BN,)
    return pl.pallas_call(
        _gates_body,
        grid=grid,
        in_specs=[_blk(128)] * 6 + [_full((768, 256)), _full((768, 128)),
                                    _full((1, 256)), _full((1, 128))],
        out_specs=[_blk(128)] * 3,
        out_shape=[jax.ShapeDtypeStruct((N, 128), jnp.float32)] * 3,
    )(p0, p1, q0, q1, s0, s1, wgT, wcT, bg, bc)


def _tc_final(z, cp, rh, t1a, t1b, u1a, u1b, hx, wcT, bc):
    grid = (N // BN,)
    return pl.pallas_call(
        _final_body,
        grid=grid,
        in_specs=[_blk(128)] * 3 + [_blk(64)] * 4 + [_blk(128),
                                                     _full((768, 128)),
                                                     _full((1, 128))],
        out_specs=_blk(128),
        out_shape=jax.ShapeDtypeStruct((N, 128), jnp.float32),
    )(z, cp, rh, t1a, t1b, u1a, u1b, hx, wcT, bc)


def kernel(inputs, hx, edge_index, edge_weight, weight_gate, weight_candidate,
           bias_gate, bias_candidate):
    p0 = inputs[0]
    p1 = hx[0]
    col3 = edge_index[1].astype(jnp.int32).reshape(NS, NCHUNK, G)
    row3 = edge_index[0].astype(jnp.int32).reshape(NS, NCHUNK, G)
    w3 = edge_weight.reshape(NS, NCHUNK, G)

    conv128 = _make_conv(128)
    conv64 = _make_conv(64)

    q0, q1 = conv128(p0, p1, col3, row3, w3)
    s0, s1 = conv128(q0, q1, col3, row3, w3)

    wgT = weight_gate.T
    wcT = weight_candidate.T
    bg = bias_gate.reshape(1, 256)
    bc = bias_candidate.reshape(1, 128)

    z, rh, cp = _tc_gates(p0, p1, q0, q1, s0, s1, wgT, wcT, bg, bc)

    c1a = rh[:, 0:64]
    c1b = rh[:, 64:128]
    t1a, t1b = conv64(c1a, c1b, col3, row3, w3)
    u1a, u1b = conv64(t1a, t1b, col3, row3, w3)

    h = _tc_final(z, cp, rh, t1a, t1b, u1a, u1b, p1, wcT, bc)
    return h[None]


# speedup vs baseline: 4.0267x; 4.0267x over previous
import functools
import jax
import jax.numpy as jnp
from jax import lax
from jax.experimental import pallas as pl
from jax.experimental.pallas import tpu as pltpu
from jax.experimental.pallas import tpu_sc as plsc

N = 10000
E = 320000
NS = 16
EPT = E // NS
G = 80
CPS = 25
SUPER = EPT // (CPS * G)
RB = 16
RPT = 624
NB = 40


def _conv_body(F, x_hbm, col_hbm, row_hbm, w_hbm, y_hbm,
               col_v, row_v, w_v, rows_v, zero_v, acc_sh, sem):
    c = lax.axis_index("c")
    s = lax.axis_index("s")
    cN = c * N

    def zbody(i, carry):
        for k in range(F // 16):
            zero_v[i, pl.ds(k * 16, 16)] = jnp.zeros((16,), jnp.float32)
        return carry
    lax.fori_loop(0, RB, zbody, 0)

    base = s * RPT

    def zcopy(i, carry):
        @pl.when((i < NB - 1) | (s == NS - 1))
        def _():
            pltpu.sync_copy(zero_v, acc_sh.at[pl.ds(base + i * RB, RB)])
        return carry
    lax.fori_loop(0, NB, zcopy, 0)
    plsc.subcore_barrier()

    def super_body(ss, carry):
        pltpu.sync_copy(col_hbm.at[s, ss], col_v)
        pltpu.sync_copy(row_hbm.at[s, ss], row_v)
        pltpu.sync_copy(w_hbm.at[s, ss], w_v)

        cNv = jnp.full((16,), cN, jnp.int32)

        def shift(i, carry2):
            for k in range(G // 16):
                col_v[i, pl.ds(k * 16, 16)] = col_v[i, pl.ds(k * 16, 16)] + cNv
            return carry2
        lax.fori_loop(0, CPS, shift, 0)

        def chunk_body(i, carry2):
            pltpu.async_copy(x_hbm.at[col_v.at[i]], rows_v, sem).wait()

            for j in range(G):
                wj = plsc.load_gather(
                    w_v, [jnp.full((16,), i * G + j, jnp.int32)])
                for k in range(F // 16):
                    rows_v[j, pl.ds(k * 16, 16)] = (
                        rows_v[j, pl.ds(k * 16, 16)] * wj)

            pltpu.sync_copy(rows_v, acc_sh.at[row_v.at[i]], add=True)
            return carry2
        lax.fori_loop(0, CPS, chunk_body, 0)
        return carry
    lax.fori_loop(0, SUPER, super_body, 0)

    plsc.subcore_barrier()

    def ocopy(i, carry):
        @pl.when((i < NB - 1) | (s == NS - 1))
        def _():
            pltpu.sync_copy(acc_sh.at[pl.ds(base + i * RB, RB)],
                            y_hbm.at[pl.ds(cN + base + i * RB, RB)])
        return carry
    lax.fori_loop(0, NB, ocopy, 0)


@functools.lru_cache(maxsize=None)
def _make_conv(F):
    mesh = plsc.VectorSubcoreMesh(core_axis_name="c", subcore_axis_name="s")
    return pl.kernel(
        functools.partial(_conv_body, F),
        out_type=jax.ShapeDtypeStruct((2 * N, F), jnp.float32),
        mesh=mesh,
        compiler_params=pltpu.CompilerParams(needs_layout_passes=False,
                                             use_tc_tiling_on_sc=False),
        scratch_types=[
            pltpu.VMEM((CPS, G), jnp.int32),
            pltpu.VMEM((CPS, G), jnp.int32),
            pltpu.VMEM((CPS * G,), jnp.float32),
            pltpu.VMEM((G, F), jnp.float32),
            pltpu.VMEM((RB, F), jnp.float32),
            pltpu.VMEM_SHARED((N, F), jnp.float32),
            pltpu.SemaphoreType.DMA,
        ],
    )


BN = 1000


def _gates_body(p0, p1, q0, q1, s0, s1, wgT, wcT, bg, bc, z_o, rh_o, cp_o):
    d4 = 2.0 * s0[...] - p0[...]
    d5 = 2.0 * s1[...] - p1[...]
    g = (jnp.dot(p0[...], wgT[0:128, :], preferred_element_type=jnp.float32)
         + jnp.dot(p1[...], wgT[128:256, :], preferred_element_type=jnp.float32)
         + jnp.dot(q0[...], wgT[256:384, :], preferred_element_type=jnp.float32)
         + jnp.dot(q1[...], wgT[384:512, :], preferred_element_type=jnp.float32)
         + jnp.dot(d4, wgT[512:640, :], preferred_element_type=jnp.float32)
         + jnp.dot(d5, wgT[640:768, :], preferred_element_type=jnp.float32)
         + bg[...])
    g = jax.nn.sigmoid(g)
    z = g[:, 0:128]
    r = g[:, 128:256]
    z_o[...] = z
    rh_o[...] = r * p1[...]
    cp_o[...] = (jnp.dot(p0[...], wcT[0:128, :], preferred_element_type=jnp.float32)
                 + jnp.dot(q0[...], wcT[256:384, :], preferred_element_type=jnp.float32)
                 + jnp.dot(d4, wcT[512:640, :], preferred_element_type=jnp.float32)
                 + bc[...])


def _final_body(z, cp, rh, t1a, t1b, u1a, u1b, hx, wcT, bc, h_o):
    rhv = rh[...]
    cand = (cp[...]
            + jnp.dot(rhv, wcT[128:256, :], preferred_element_type=jnp.float32)
            + jnp.dot(t1a[...], wcT[384:448, :], preferred_element_type=jnp.float32)
            + jnp.dot(t1b[...], wcT[448:512, :], preferred_element_type=jnp.float32)
            + jnp.dot(2.0 * u1a[...] - rhv[:, 0:64], wcT[640:704, :],
                      preferred_element_type=jnp.float32)
            + jnp.dot(2.0 * u1b[...] - rhv[:, 64:128], wcT[704:768, :],
                      preferred_element_type=jnp.float32))
    cand = jnp.tanh(cand)
    zv = z[...]
    h_o[...] = (1.0 - zv) * hx[...] + zv * cand


def _blk(fw):
    return pl.BlockSpec((BN, fw), lambda i: (i, 0))


def _blk_hi(fw):
    return pl.BlockSpec((BN, fw), lambda i: (i + N // BN, 0))


def _full(shape):
    return pl.BlockSpec(shape, lambda i: tuple(0 for _ in shape))


def _tc_gates(p0, p1, Y1, Y2, wgT, wcT, bg, bc):
    grid = (N // BN,)
    return pl.pallas_call(
        _gates_body,
        grid=grid,
        in_specs=[_blk(128), _blk(128), _blk(128), _blk_hi(128),
                  _blk(128), _blk_hi(128),
                  _full((768, 256)), _full((768, 128)),
                  _full((1, 256)), _full((1, 128))],
        out_specs=[_blk(128)] * 3,
        out_shape=[jax.ShapeDtypeStruct((N, 128), jnp.float32)] * 3,
    )(p0, p1, Y1, Y1, Y2, Y2, wgT, wcT, bg, bc)


def _tc_final(z, cp, rh, Y3, Y4, hx, wcT, bc):
    grid = (N // BN,)
    return pl.pallas_call(
        _final_body,
        grid=grid,
        in_specs=[_blk(128)] * 3 + [_blk(64), _blk_hi(64), _blk(64),
                                    _blk_hi(64), _blk(128),
                                    _full((768, 128)), _full((1, 128))],
        out_specs=_blk(128),
        out_shape=jax.ShapeDtypeStruct((N, 128), jnp.float32),
    )(z, cp, rh, Y3, Y3, Y4, Y4, hx, wcT, bc)


def kernel(inputs, hx, edge_index, edge_weight, weight_gate, weight_candidate,
           bias_gate, bias_candidate):
    p0 = inputs[0]
    p1 = hx[0]
    col3 = edge_index[1].astype(jnp.int32).reshape(NS, SUPER, CPS, G)
    row3 = edge_index[0].astype(jnp.int32).reshape(NS, SUPER, CPS, G)
    w3 = edge_weight.reshape(NS, SUPER, CPS * G)

    conv128 = _make_conv(128)
    conv64 = _make_conv(64)

    X1 = jnp.concatenate([p0, p1], axis=0)
    Y1 = conv128(X1, col3, row3, w3)
    Y2 = conv128(Y1, col3, row3, w3)

    wgT = weight_gate.T
    wcT = weight_candidate.T
    bg = bias_gate.reshape(1, 256)
    bc = bias_candidate.reshape(1, 128)

    z, rh, cp = _tc_gates(p0, p1, Y1, Y2, wgT, wcT, bg, bc)

    rh_st = jnp.concatenate([rh[:, 0:64], rh[:, 64:128]], axis=0)
    Y3 = conv64(rh_st, col3, row3, w3)
    Y4 = conv64(Y3, col3, row3, w3)

    h = _tc_final(z, cp, rh, Y3, Y4, p1, wcT, bc)
    return h[None]
